# Initial kernel scaffold; baseline (speedup 1.0000x reference)
import jax
import jax.numpy as jnp
from jax.experimental import pallas as pl


def kernel(mem, idx, val):
    raise NotImplementedError("write your pallas kernel here")


# speedup vs baseline: 13.6012x; 13.6012x over previous
import jax
import jax.numpy as jnp
from jax import lax
from jax.experimental import pallas as pl
from jax.experimental.pallas import tpu as pltpu
from jax.experimental.pallas import tpu_sc as plsc

C = 8
N = 128 ** 3
B = N // 4
DECAY = 0.95

NP = 4
JQ = B // NP
SUB = 8192
NBLK = JQ // SUB
NBKT = 64
CAP = 192

SLAB = 32768
NSLAB = 16
QN = N // 4


def _body(mem_hbm, idx_hbm, val_hbm,
          out_hbm, tbin_hbm, vbin_hbm, cnt_hbm,
          ibuf, vbuf, stg_t, stg_v, cnts, mslab, vslab, bint, binv, cntp,
          dsem):
    wid = lax.axis_index("core") * 16 + lax.axis_index("sub")
    c = wid // 4
    jq = wid % 4
    q = jq
    p = wid

    iota = lax.iota(jnp.int32, 16)
    bse, _ = plsc.scan_count(jnp.zeros((16,), jnp.int32))
    base_v = jnp.full((16,), bse[0], jnp.int32)

    m32767 = jnp.full((16,), 32767, jnp.int32)
    capm1 = jnp.full((16,), CAP - 1, jnp.int32)
    capv = jnp.full((16,), CAP, jnp.int32)
    one = jnp.full((16,), 1, jnp.int32)
    zi16 = jnp.zeros((16,), jnp.int32)

    def subblock(blk, _):
        j0 = jq * JQ + blk * SUB
        pltpu.sync_copy(idx_hbm.at[c, pl.ds(j0, SUB)], ibuf)
        pltpu.sync_copy(val_hbm.at[c, pl.ds(j0, SUB)], vbuf)

        def rst(g, _):
            cnts[pl.ds(g * 16, 16)] = zi16
            return 0
        lax.fori_loop(0, NBKT // 16, rst, 0)

        def binify(i, _):
            iv = ibuf[pl.ds(i * 16, 16)]
            vv = vbuf[pl.ds(i * 16, 16)]
            bkt = lax.shift_right_logical(iv, 15)
            tloc = lax.bitwise_and(iv, m32767)
            cur = plsc.load_gather(cnts, [bkt])
            rank, lastm = plsc.scan_count(bkt)
            rank0 = rank - base_v
            pos = jnp.minimum(cur + rank0, capm1)
            plsc.store_scatter(stg_t, [bkt, pos], tloc)
            plsc.store_scatter(stg_v, [bkt, pos], vv)
            plsc.store_scatter(cnts, [bkt], jnp.minimum(cur + rank0 + one, capv),
                               mask=lastm)
            return 0

        lax.fori_loop(0, SUB // 16, binify, 0)

        pltpu.sync_copy(stg_t, tbin_hbm.at[p, blk])
        pltpu.sync_copy(stg_v, vbin_hbm.at[p, blk])
        pltpu.sync_copy(cnts, cnt_hbm.at[p, blk])
        return 0

    lax.fori_loop(0, NBLK, subblock, 0)

    plsc.subcore_barrier()

    def ldcnt(pp, _):
        pltpu.sync_copy(cnt_hbm.at[c * 4 + pp], cntp.at[pp])
        return 0
    lax.fori_loop(0, NP, ldcnt, 0)

    def slab_loop(s, _):
        bkt = q * NSLAB + s
        cell0 = q * QN + s * SLAB

        sent = jnp.full((16,), -1.0, jnp.float32)

        def init(i, _):
            vslab[pl.ds(i * 16, 16)] = sent
            return 0
        lax.fori_loop(0, SLAB // 16, init, 0)

        def prod_loop(pp, _):
            def blk_loop(blk, _):
                pltpu.make_async_copy(
                    tbin_hbm.at[c * 4 + pp, blk, bkt], bint, dsem).start()
                pltpu.make_async_copy(
                    vbin_hbm.at[c * 4 + pp, blk, bkt], binv, dsem).start()
                pltpu.make_async_copy(
                    tbin_hbm.at[c * 4 + pp, blk, bkt], bint, dsem).wait()
                pltpu.make_async_copy(
                    vbin_hbm.at[c * 4 + pp, blk, bkt], binv, dsem).wait()
                cnt_l = plsc.load_gather(
                    cntp, [jnp.full((16,), pp, jnp.int32),
                           jnp.full((16,), blk, jnp.int32),
                           jnp.full((16,), bkt, jnp.int32)])

                def apply(v, _):
                    tq = bint[pl.ds(v * 16, 16)]
                    vq = binv[pl.ds(v * 16, 16)]
                    inb = (jnp.full((16,), v * 16, jnp.int32) + iota) < cnt_l
                    vqm = jnp.where(inb, vq, jnp.full((16,), -1.0, jnp.float32))
                    vs, ts = plsc.sort_key_val(
                        vqm, lax.bitwise_and(tq, m32767))
                    valid = vs >= 0.0
                    _, lastm = plsc.scan_count(ts, valid)
                    old = plsc.load_gather(vslab, [ts])
                    plsc.store_scatter(vslab, [ts], jnp.maximum(old, vs),
                                       mask=valid & lastm)
                    return 0

                lax.fori_loop(0, CAP // 16, apply, 0)
                return 0

            lax.fori_loop(0, NBLK, blk_loop, 0)
            return 0

        lax.fori_loop(0, NP, prod_loop, 0)

        pltpu.sync_copy(mem_hbm.at[c, pl.ds(cell0, SLAB)], mslab)

        def merge(i, _):
            m = mslab[pl.ds(i * 16, 16)]
            tv = vslab[pl.ds(i * 16, 16)]
            mslab[pl.ds(i * 16, 16)] = jnp.where(
                m < 0.0, m, jnp.maximum(m * DECAY, tv))
            return 0

        lax.fori_loop(0, SLAB // 16, merge, 0)
        pltpu.sync_copy(mslab, out_hbm.at[c, pl.ds(cell0, SLAB)])
        return 0

    lax.fori_loop(0, NSLAB, slab_loop, 0)


@jax.jit
def kernel(mem, idx, val):
    idx = idx.astype(jnp.int32)
    run = pl.kernel(
        _body,
        out_type=(
            jax.ShapeDtypeStruct((C, N), jnp.float32),
            jax.ShapeDtypeStruct((32, NBLK, NBKT, CAP), jnp.int32),
            jax.ShapeDtypeStruct((32, NBLK, NBKT, CAP), jnp.float32),
            jax.ShapeDtypeStruct((32, NBLK, NBKT), jnp.int32),
        ),
        mesh=plsc.VectorSubcoreMesh(
            core_axis_name="core", subcore_axis_name="sub"),
        compiler_params=pltpu.CompilerParams(needs_layout_passes=False),
        scratch_types=[
            pltpu.VMEM((SUB,), jnp.int32),
            pltpu.VMEM((SUB,), jnp.float32),
            pltpu.VMEM((NBKT, CAP), jnp.int32),
            pltpu.VMEM((NBKT, CAP), jnp.float32),
            pltpu.VMEM((NBKT,), jnp.int32),
            pltpu.VMEM((SLAB,), jnp.float32),
            pltpu.VMEM((SLAB,), jnp.float32),
            pltpu.VMEM((CAP,), jnp.int32),
            pltpu.VMEM((CAP,), jnp.float32),
            pltpu.VMEM((NP, NBLK, NBKT), jnp.int32),
            pltpu.SemaphoreType.DMA,
        ],
    )
    return run(mem, idx, val)[0]


# speedup vs baseline: 14.7469x; 1.0842x over previous
import jax
import jax.numpy as jnp
from jax import lax
from jax.experimental import pallas as pl
from jax.experimental.pallas import tpu as pltpu
from jax.experimental.pallas import tpu_sc as plsc

C = 8
N = 128 ** 3
B = N // 4
DECAY = 0.95

NP = 4
JQ = B // NP
SUB = 8192
NBLK = JQ // SUB
NBKT = 64
CAP = 192

SLAB = 32768
NSLAB = 16
QN = N // 4


def _body(mem_hbm, idx_hbm, val_hbm,
          out_hbm, tbin_hbm, vbin_hbm, cnt_hbm,
          ibuf, vbuf, stg_t, stg_v, cnts, mslab, vslab, bint, binv, cntp,
          dsem):
    wid = lax.axis_index("core") * 16 + lax.axis_index("sub")
    c = wid // 4
    jq = wid % 4
    q = jq
    p = wid

    iota = lax.iota(jnp.int32, 16)
    bse, _ = plsc.scan_count(jnp.zeros((16,), jnp.int32))
    base_v = jnp.full((16,), bse[0], jnp.int32)

    m32767 = jnp.full((16,), 32767, jnp.int32)
    capm1 = jnp.full((16,), CAP - 1, jnp.int32)
    capv = jnp.full((16,), CAP, jnp.int32)
    one = jnp.full((16,), 1, jnp.int32)
    zi16 = jnp.zeros((16,), jnp.int32)

    def subblock(blk, _):
        j0 = jq * JQ + blk * SUB
        pltpu.sync_copy(idx_hbm.at[c, pl.ds(j0, SUB)], ibuf)
        pltpu.sync_copy(val_hbm.at[c, pl.ds(j0, SUB)], vbuf)

        def rst(g, _):
            cnts[pl.ds(g * 16, 16)] = zi16
            return 0
        lax.fori_loop(0, NBKT // 16, rst, 0)

        def binify(i, _):
            iv = ibuf[pl.ds(i * 16, 16)]
            vv = vbuf[pl.ds(i * 16, 16)]
            bkt = lax.shift_right_logical(iv, 15)
            tloc = lax.bitwise_and(iv, m32767)
            cur = plsc.load_gather(cnts, [bkt])
            rank, lastm = plsc.scan_count(bkt)
            rank0 = rank - base_v
            pos = jnp.minimum(cur + rank0, capm1)
            plsc.store_scatter(stg_t, [bkt, pos], tloc)
            plsc.store_scatter(stg_v, [bkt, pos], vv)
            plsc.store_scatter(cnts, [bkt], jnp.minimum(cur + rank0 + one, capv),
                               mask=lastm)
            return 0

        lax.fori_loop(0, SUB // 16, binify, 0)

        pltpu.sync_copy(stg_t, tbin_hbm.at[p, blk])
        pltpu.sync_copy(stg_v, vbin_hbm.at[p, blk])
        pltpu.sync_copy(cnts, cnt_hbm.at[p, blk])
        return 0

    lax.fori_loop(0, NBLK, subblock, 0)

    plsc.subcore_barrier()

    def ldcnt(pp, _):
        pltpu.sync_copy(cnt_hbm.at[c * 4 + pp], cntp.at[pp])
        return 0
    lax.fori_loop(0, NP, ldcnt, 0)

    def slab_loop(s, _):
        bkt = q * NSLAB + s
        cell0 = q * QN + s * SLAB

        sent = jnp.full((16,), -1.0, jnp.float32)

        def init(i, _):
            vslab[pl.ds(i * 16, 16)] = sent
            return 0
        lax.fori_loop(0, SLAB // 16, init, 0)

        def prod_loop(pp, _):
            def blk_loop(blk, _):
                pltpu.make_async_copy(
                    tbin_hbm.at[c * 4 + pp, blk, bkt], bint, dsem).start()
                pltpu.make_async_copy(
                    vbin_hbm.at[c * 4 + pp, blk, bkt], binv, dsem).start()
                pltpu.make_async_copy(
                    tbin_hbm.at[c * 4 + pp, blk, bkt], bint, dsem).wait()
                pltpu.make_async_copy(
                    vbin_hbm.at[c * 4 + pp, blk, bkt], binv, dsem).wait()
                cnt_l = plsc.load_gather(
                    cntp, [jnp.full((16,), pp, jnp.int32),
                           jnp.full((16,), blk, jnp.int32),
                           jnp.full((16,), bkt, jnp.int32)])

                def apply(v, _):
                    tq = bint[pl.ds(v * 16, 16)]
                    vq = binv[pl.ds(v * 16, 16)]
                    inb = (jnp.full((16,), v * 16, jnp.int32) + iota) < cnt_l
                    tqc = lax.bitwise_and(tq, m32767)
                    _, lastm = plsc.scan_count(tqc, inb)
                    plsc.store_scatter(vslab, [tqc], vq, mask=inb & lastm)
                    return 0

                lax.fori_loop(0, CAP // 16, apply, 0)
                return 0

            lax.fori_loop(0, NBLK, blk_loop, 0)
            return 0

        lax.fori_loop(0, NP, prod_loop, 0)

        pltpu.sync_copy(mem_hbm.at[c, pl.ds(cell0, SLAB)], mslab)

        def merge(i, _):
            m = mslab[pl.ds(i * 16, 16)]
            tv = vslab[pl.ds(i * 16, 16)]
            mslab[pl.ds(i * 16, 16)] = jnp.where(
                m < 0.0, m, jnp.maximum(m * DECAY, tv))
            return 0

        lax.fori_loop(0, SLAB // 16, merge, 0)
        pltpu.sync_copy(mslab, out_hbm.at[c, pl.ds(cell0, SLAB)])
        return 0

    lax.fori_loop(0, NSLAB, slab_loop, 0)


@jax.jit
def kernel(mem, idx, val):
    idx = idx.astype(jnp.int32)
    run = pl.kernel(
        _body,
        out_type=(
            jax.ShapeDtypeStruct((C, N), jnp.float32),
            jax.ShapeDtypeStruct((32, NBLK, NBKT, CAP), jnp.int32),
            jax.ShapeDtypeStruct((32, NBLK, NBKT, CAP), jnp.float32),
            jax.ShapeDtypeStruct((32, NBLK, NBKT), jnp.int32),
        ),
        mesh=plsc.VectorSubcoreMesh(
            core_axis_name="core", subcore_axis_name="sub"),
        compiler_params=pltpu.CompilerParams(needs_layout_passes=False),
        scratch_types=[
            pltpu.VMEM((SUB,), jnp.int32),
            pltpu.VMEM((SUB,), jnp.float32),
            pltpu.VMEM((NBKT, CAP), jnp.int32),
            pltpu.VMEM((NBKT, CAP), jnp.float32),
            pltpu.VMEM((NBKT,), jnp.int32),
            pltpu.VMEM((SLAB,), jnp.float32),
            pltpu.VMEM((SLAB,), jnp.float32),
            pltpu.VMEM((CAP,), jnp.int32),
            pltpu.VMEM((CAP,), jnp.float32),
            pltpu.VMEM((NP, NBLK, NBKT), jnp.int32),
            pltpu.SemaphoreType.DMA,
        ],
    )
    return run(mem, idx, val)[0]
